# Initial kernel scaffold; baseline (speedup 1.0000x reference)
import jax
import jax.numpy as jnp
from jax.experimental import pallas as pl


def kernel(ex_raw, in_raw, o_pre, gj_src, gj_dst, gj_w, threshold, decay):
    raise NotImplementedError("write your pallas kernel here")


# speedup vs baseline: 35.4402x; 35.4402x over previous
import functools

import jax
import jax.numpy as jnp
from jax import lax
from jax.experimental import pallas as pl
from jax.experimental.pallas import tpu as pltpu
from jax.experimental.pallas import tpu_sc as plsc

N = 100000
E = 3200000
B = 8

NC = 2
NS = 16
NW = NC * NS
E_PER_W = E // NW
G = 80
CHUNK = 4000
GROUPS = CHUNK // G
NCHUNK = E_PER_W // CHUNK
ROWS_PER_TILE = N // NS


@functools.cache
def _make_sc_kernel():
    mesh = plsc.VectorSubcoreMesh(core_axis_name="c", subcore_axis_name="s",
                                  num_cores=NC, num_subcores=NS)

    @functools.partial(
        pl.kernel,
        out_type=jax.ShapeDtypeStruct((NC, N, B), jnp.float32),
        mesh=mesh,
        scratch_types=[
            pltpu.VMEM((GROUPS, G), jnp.int32),
            pltpu.VMEM((GROUPS, G), jnp.int32),
            pltpu.VMEM((CHUNK,), jnp.float32),
            pltpu.VMEM((CHUNK, B), jnp.float32),
            pltpu.VMEM_SHARED((N, B), jnp.float32),
            pltpu.SemaphoreType.DMA,
        ],
        compiler_params=pltpu.CompilerParams(use_tc_tiling_on_sc=False,
                                             needs_layout_passes=False),
    )
    def sc_kernel(o_t, src2, dst2, w_hbm, zeros_hbm, out_hbm,
                  src_v, dst_v, w_v, rows_v, acc, sem):
        cid = lax.axis_index("c")
        sid = lax.axis_index("s")
        wid = cid * NS + sid

        pltpu.sync_copy(zeros_hbm.at[pl.ds(sid * ROWS_PER_TILE, ROWS_PER_TILE)],
                        acc.at[pl.ds(sid * ROWS_PER_TILE, ROWS_PER_TILE)])
        plsc.subcore_barrier()

        lanes = lax.iota(jnp.int32, 16)
        sub = lanes >> 3
        col = lanes & 7

        row_base = wid * (E_PER_W // G)

        def chunk_body(c, carry):
            r0 = row_base + c * GROUPS
            pltpu.sync_copy(src2.at[pl.ds(r0, GROUPS)], src_v)
            pltpu.sync_copy(dst2.at[pl.ds(r0, GROUPS)], dst_v)
            pltpu.sync_copy(w_hbm.at[pl.ds(wid * E_PER_W + c * CHUNK, CHUNK)], w_v)

            descs = []
            for g in range(GROUPS):
                descs.append(pltpu.async_copy(
                    o_t.at[src_v.at[g]], rows_v.at[pl.ds(g * G, G)], sem))
            for d in descs:
                d.wait()

            def vbody(i, carry2):
                ridx = i * 2 + sub
                vals = plsc.load_gather(rows_v, [ridx, col])
                wv = plsc.load_gather(w_v, [ridx])
                plsc.store_scatter(rows_v, [ridx, col], jnp.abs(vals) * wv)
                return carry2
            lax.fori_loop(0, CHUNK * B // 16, vbody, 0)

            for g in range(GROUPS):
                pltpu.sync_copy(rows_v.at[pl.ds(g * G, G)],
                                acc.at[dst_v.at[g]], add=True)
            return carry

        lax.fori_loop(0, NCHUNK, chunk_body, 0)

        plsc.subcore_barrier()
        pltpu.sync_copy(acc.at[pl.ds(sid * ROWS_PER_TILE, ROWS_PER_TILE)],
                        out_hbm.at[cid, pl.ds(sid * ROWS_PER_TILE, ROWS_PER_TILE)])

    return sc_kernel


_EPI_ROWS = 625
_EPI_COLS = (N * B) // _EPI_ROWS


def _epi_body(ex_ref, in_ref, a0_ref, a1_ref, thr_ref, o_ref):
    s = jnp.clip(ex_ref[...] - in_ref[...] + a0_ref[...] + a1_ref[...],
                 -10.0, 10.0)
    o_ref[...] = jnp.maximum(s - jax.nn.softplus(thr_ref[...]), 0.0)


def _epilogue(ex_t, in_t, a0, a1, thr8):
    spec = pl.BlockSpec((_EPI_ROWS, 128), lambda i: (0, i))
    return pl.pallas_call(
        _epi_body,
        out_shape=jax.ShapeDtypeStruct((_EPI_ROWS, _EPI_COLS), jnp.float32),
        grid=(_EPI_COLS // 128,),
        in_specs=[spec] * 5,
        out_specs=spec,
    )(ex_t, in_t, a0, a1, thr8)


@jax.jit
def kernel(ex_raw, in_raw, o_pre, gj_src, gj_dst, gj_w, threshold, decay):
    del decay
    o_t = o_pre.T
    src2 = gj_src.reshape(E // G, G)
    dst2 = gj_dst.reshape(E // G, G)
    zeros8 = jnp.zeros((N, B), jnp.float32)
    acc = _make_sc_kernel()(o_t, src2, dst2, gj_w, zeros8)

    ex_t = ex_raw.T.reshape(_EPI_ROWS, _EPI_COLS)
    in_t = in_raw.T.reshape(_EPI_ROWS, _EPI_COLS)
    thr8 = jnp.repeat(threshold, B).reshape(_EPI_ROWS, _EPI_COLS)
    a0 = acc[0].reshape(_EPI_ROWS, _EPI_COLS)
    a1 = acc[1].reshape(_EPI_ROWS, _EPI_COLS)
    out_t = _epilogue(ex_t, in_t, a0, a1, thr8)
    return out_t.reshape(N, B).T


# speedup vs baseline: 41.2334x; 1.1635x over previous
import functools

import jax
import jax.numpy as jnp
from jax import lax
from jax.experimental import pallas as pl
from jax.experimental.pallas import tpu as pltpu
from jax.experimental.pallas import tpu_sc as plsc

N = 100000
E = 3200000
B = 8

NC = 2
NS = 16
NW = NC * NS
E_PER_W = E // NW
G = 80
CHUNK = 4000
GROUPS = CHUNK // G
NCHUNK = E_PER_W // CHUNK
ROWS_PER_TILE = N // NS


@functools.cache
def _make_sc_kernel():
    mesh = plsc.VectorSubcoreMesh(core_axis_name="c", subcore_axis_name="s",
                                  num_cores=NC, num_subcores=NS)

    @functools.partial(
        pl.kernel,
        out_type=jax.ShapeDtypeStruct((NC, N, B), jnp.float32),
        mesh=mesh,
        scratch_types=[
            pltpu.VMEM((GROUPS, G), jnp.int32),
            pltpu.VMEM((GROUPS, G), jnp.int32),
            pltpu.VMEM((CHUNK,), jnp.float32),
            pltpu.VMEM((CHUNK, B), jnp.float32),
            pltpu.VMEM_SHARED((N, B), jnp.float32),
            pltpu.SemaphoreType.DMA,
            pltpu.SemaphoreType.DMA,
        ],
        compiler_params=pltpu.CompilerParams(use_tc_tiling_on_sc=False,
                                             needs_layout_passes=False),
    )
    def sc_kernel(o_t, src2, dst2, w_hbm, zeros_hbm, out_hbm,
                  src_v, dst_v, w_v, rows_v, acc, sem, ssem):
        cid = lax.axis_index("c")
        sid = lax.axis_index("s")
        wid = cid * NS + sid

        pltpu.sync_copy(zeros_hbm.at[pl.ds(sid * ROWS_PER_TILE, ROWS_PER_TILE)],
                        acc.at[pl.ds(sid * ROWS_PER_TILE, ROWS_PER_TILE)])
        plsc.subcore_barrier()

        lanes = lax.iota(jnp.int32, 16)
        sub = lanes >> 3
        col = lanes & 7

        row_base = wid * (E_PER_W // G)

        def chunk_body(c, carry):
            r0 = row_base + c * GROUPS
            e_descs = [
                pltpu.async_copy(src2.at[pl.ds(r0, GROUPS)], src_v, sem),
                pltpu.async_copy(dst2.at[pl.ds(r0, GROUPS)], dst_v, sem),
                pltpu.async_copy(
                    w_hbm.at[pl.ds(wid * E_PER_W + c * CHUNK, CHUNK)], w_v, sem),
            ]
            for d in e_descs:
                d.wait()

            descs = []
            for g in range(GROUPS):
                descs.append(pltpu.async_copy(
                    o_t.at[src_v.at[g]], rows_v.at[pl.ds(g * G, G)], sem))
            for d in descs:
                d.wait()

            def vbody(i, carry2):
                e0 = i * 16
                for j in range(8):
                    ridx = e0 + 2 * j + sub
                    vals = plsc.load_gather(rows_v, [ridx, col])
                    wv = plsc.load_gather(w_v, [ridx])
                    plsc.store_scatter(rows_v, [ridx, col], jnp.abs(vals) * wv)
                return carry2
            lax.fori_loop(0, CHUNK // 16, vbody, 0)

            s_descs = []
            for g in range(GROUPS):
                s_descs.append(pltpu.async_copy(
                    rows_v.at[pl.ds(g * G, G)], acc.at[dst_v.at[g]],
                    ssem, add=True))
            for d in s_descs:
                d.wait()
            return carry

        lax.fori_loop(0, NCHUNK, chunk_body, 0)

        plsc.subcore_barrier()
        pltpu.sync_copy(acc.at[pl.ds(sid * ROWS_PER_TILE, ROWS_PER_TILE)],
                        out_hbm.at[cid, pl.ds(sid * ROWS_PER_TILE, ROWS_PER_TILE)])

    return sc_kernel


_EPI_ROWS = 625
_EPI_COLS = (N * B) // _EPI_ROWS


def _epi_body(ex_ref, in_ref, a0_ref, a1_ref, thr_ref, o_ref):
    s = jnp.clip(ex_ref[...] - in_ref[...] + a0_ref[...] + a1_ref[...],
                 -10.0, 10.0)
    o_ref[...] = jnp.maximum(s - jax.nn.softplus(thr_ref[...]), 0.0)


def _epilogue(ex_t, in_t, a0, a1, thr8):
    spec = pl.BlockSpec((_EPI_ROWS, 128), lambda i: (0, i))
    return pl.pallas_call(
        _epi_body,
        out_shape=jax.ShapeDtypeStruct((_EPI_ROWS, _EPI_COLS), jnp.float32),
        grid=(_EPI_COLS // 128,),
        in_specs=[spec] * 5,
        out_specs=spec,
    )(ex_t, in_t, a0, a1, thr8)


@jax.jit
def kernel(ex_raw, in_raw, o_pre, gj_src, gj_dst, gj_w, threshold, decay):
    del decay
    o_t = o_pre.T
    src2 = gj_src.reshape(E // G, G)
    dst2 = gj_dst.reshape(E // G, G)
    zeros8 = jnp.zeros((N, B), jnp.float32)
    acc = _make_sc_kernel()(o_t, src2, dst2, gj_w, zeros8)

    ex_t = ex_raw.T.reshape(_EPI_ROWS, _EPI_COLS)
    in_t = in_raw.T.reshape(_EPI_ROWS, _EPI_COLS)
    thr8 = jnp.repeat(threshold, B).reshape(_EPI_ROWS, _EPI_COLS)
    a0 = acc[0].reshape(_EPI_ROWS, _EPI_COLS)
    a1 = acc[1].reshape(_EPI_ROWS, _EPI_COLS)
    out_t = _epilogue(ex_t, in_t, a0, a1, thr8)
    return out_t.reshape(N, B).T


# speedup vs baseline: 51.1560x; 1.2406x over previous
import functools

import jax
import jax.numpy as jnp
from jax import lax
from jax.experimental import pallas as pl
from jax.experimental.pallas import tpu as pltpu
from jax.experimental.pallas import tpu_sc as plsc

N = 100000
E = 3200000
B = 8

NC = 2
NS = 16
NW = NC * NS
E_PER_W = E // NW
G = 80
CHUNK = 4000
GROUPS = CHUNK // G
NCHUNK = E_PER_W // CHUNK
ROWS_PER_TILE = N // NS


@functools.cache
def _make_sc_kernel():
    mesh = plsc.VectorSubcoreMesh(core_axis_name="c", subcore_axis_name="s",
                                  num_cores=NC, num_subcores=NS)

    @functools.partial(
        pl.kernel,
        out_type=jax.ShapeDtypeStruct((NC, N, B), jnp.float32),
        mesh=mesh,
        scratch_types=[
            pltpu.VMEM((GROUPS, G), jnp.int32),
            pltpu.VMEM((GROUPS, G), jnp.int32),
            pltpu.VMEM((CHUNK,), jnp.float32),
            pltpu.VMEM((CHUNK, B), jnp.float32),
            pltpu.VMEM_SHARED((N, B), jnp.float32),
            pltpu.SemaphoreType.DMA,
            pltpu.SemaphoreType.DMA,
        ],
        compiler_params=pltpu.CompilerParams(use_tc_tiling_on_sc=False,
                                             needs_layout_passes=False),
    )
    def sc_kernel(o_t, src2, dst2, w_hbm, zeros_hbm, out_hbm,
                  src_v, dst_v, w_v, rows_v, acc, sem, ssem):
        cid = lax.axis_index("c")
        sid = lax.axis_index("s")
        wid = cid * NS + sid

        pltpu.sync_copy(zeros_hbm.at[pl.ds(sid * ROWS_PER_TILE, ROWS_PER_TILE)],
                        acc.at[pl.ds(sid * ROWS_PER_TILE, ROWS_PER_TILE)])
        plsc.subcore_barrier()

        lanes = lax.iota(jnp.int32, 16)
        sub = lanes >> 3
        col = lanes & 7

        row_base = wid * (E_PER_W // G)

        def chunk_body(c, carry):
            r0 = row_base + c * GROUPS
            e_descs = [
                pltpu.async_copy(src2.at[pl.ds(r0, GROUPS)], src_v, sem),
                pltpu.async_copy(dst2.at[pl.ds(r0, GROUPS)], dst_v, sem),
                pltpu.async_copy(
                    w_hbm.at[pl.ds(wid * E_PER_W + c * CHUNK, CHUNK)], w_v, sem),
            ]
            for d in e_descs:
                d.wait()

            descs = []
            for g in range(GROUPS):
                descs.append(pltpu.async_copy(
                    o_t.at[src_v.at[g]], rows_v.at[pl.ds(g * G, G)], sem))
            for d in descs:
                d.wait()

            def vbody(i, carry2):
                e0 = i * 16
                for j in range(8):
                    ridx = e0 + 2 * j + sub
                    vals = plsc.load_gather(rows_v, [ridx, col])
                    wv = plsc.load_gather(w_v, [ridx])
                    plsc.store_scatter(rows_v, [ridx, col], jnp.abs(vals) * wv)
                return carry2
            lax.fori_loop(0, CHUNK // 16, vbody, 0)

            s_descs = []
            for g in range(GROUPS):
                s_descs.append(pltpu.async_copy(
                    rows_v.at[pl.ds(g * G, G)], acc.at[dst_v.at[g]],
                    ssem, add=True))
            for d in s_descs:
                d.wait()
            return carry

        lax.fori_loop(0, NCHUNK, chunk_body, 0)

        plsc.subcore_barrier()
        pltpu.sync_copy(acc.at[pl.ds(sid * ROWS_PER_TILE, ROWS_PER_TILE)],
                        out_hbm.at[cid, pl.ds(sid * ROWS_PER_TILE, ROWS_PER_TILE)])

    return sc_kernel


_EPI_BLK = 2048


def _epi_body(ex_ref, in_ref, acc_ref, thr_ref, o_ref):
    a = acc_ref[...]
    gj = (a[0] + a[1]).T
    s = jnp.clip(ex_ref[...] - in_ref[...] + gj, -10.0, 10.0)
    o_ref[...] = jnp.maximum(s - jax.nn.softplus(thr_ref[...]), 0.0)


def _epilogue(ex_raw, in_raw, acc, thr):
    nspec = pl.BlockSpec((B, _EPI_BLK), lambda i: (0, i))
    return pl.pallas_call(
        _epi_body,
        out_shape=jax.ShapeDtypeStruct((B, N), jnp.float32),
        grid=(pl.cdiv(N, _EPI_BLK),),
        in_specs=[
            nspec,
            nspec,
            pl.BlockSpec((NC, _EPI_BLK, B), lambda i: (0, i, 0)),
            pl.BlockSpec((1, _EPI_BLK), lambda i: (0, i)),
        ],
        out_specs=nspec,
    )(ex_raw, in_raw, acc, thr)


@jax.jit
def kernel(ex_raw, in_raw, o_pre, gj_src, gj_dst, gj_w, threshold, decay):
    del decay
    o_t = o_pre.T
    src2 = gj_src.reshape(E // G, G)
    dst2 = gj_dst.reshape(E // G, G)
    zeros8 = jnp.zeros((N, B), jnp.float32)
    acc = _make_sc_kernel()(o_t, src2, dst2, gj_w, zeros8)

    return _epilogue(ex_raw, in_raw, acc, threshold.reshape(1, N))


# speedup vs baseline: 56.2587x; 1.0997x over previous
import functools

import jax
import jax.numpy as jnp
from jax import lax
from jax.experimental import pallas as pl
from jax.experimental.pallas import tpu as pltpu
from jax.experimental.pallas import tpu_sc as plsc

N = 100000
E = 3200000
B = 8

NC = 2
NS = 16
NW = NC * NS
E_PER_W = E // NW
G = 80
CHUNK = 2000
GROUPS = CHUNK // G
NCHUNK = E_PER_W // CHUNK
NPAIR = NCHUNK // 2
ROWS_PER_TILE = N // NS


@functools.cache
def _make_sc_kernel():
    mesh = plsc.VectorSubcoreMesh(core_axis_name="c", subcore_axis_name="s",
                                  num_cores=NC, num_subcores=NS)

    @functools.partial(
        pl.kernel,
        out_type=jax.ShapeDtypeStruct((NC, N, B), jnp.float32),
        mesh=mesh,
        scratch_types=(
            [pltpu.VMEM((GROUPS, G), jnp.int32),
             pltpu.VMEM((GROUPS, G), jnp.int32),
             pltpu.VMEM((CHUNK,), jnp.float32),
             pltpu.VMEM((CHUNK, B), jnp.float32)] * 2
            + [pltpu.VMEM_SHARED((N, B), jnp.float32)]
            + [pltpu.SemaphoreType.DMA] * 6
        ),
        compiler_params=pltpu.CompilerParams(use_tc_tiling_on_sc=False,
                                             needs_layout_passes=False),
    )
    def sc_kernel(o_t, src2, dst2, w_hbm, zeros_hbm, out_hbm,
                  src_v0, dst_v0, w_v0, rows_v0,
                  src_v1, dst_v1, w_v1, rows_v1,
                  acc, esem0, esem1, gsem0, gsem1, ssem0, ssem1):
        cid = lax.axis_index("c")
        sid = lax.axis_index("s")
        wid = cid * NS + sid

        pltpu.sync_copy(zeros_hbm.at[pl.ds(sid * ROWS_PER_TILE, ROWS_PER_TILE)],
                        acc.at[pl.ds(sid * ROWS_PER_TILE, ROWS_PER_TILE)])
        plsc.subcore_barrier()

        lanes = lax.iota(jnp.int32, 16)
        sub = lanes >> 3
        col = lanes & 7

        row_base = wid * (E_PER_W // G)

        def fire_edges(c, src_v, dst_v, w_v, esem):
            r0 = row_base + c * GROUPS
            return [
                pltpu.async_copy(src2.at[pl.ds(r0, GROUPS)], src_v, esem),
                pltpu.async_copy(dst2.at[pl.ds(r0, GROUPS)], dst_v, esem),
                pltpu.async_copy(
                    w_hbm.at[pl.ds(wid * E_PER_W + c * CHUNK, CHUNK)],
                    w_v, esem),
            ]

        def fire_gathers(src_v, rows_v, gsem):
            return [pltpu.async_copy(o_t.at[src_v.at[g]],
                                     rows_v.at[pl.ds(g * G, G)], gsem)
                    for g in range(GROUPS)]

        def compute(w_v, rows_v):
            def vbody(i, carry2):
                e0 = i * 16
                for j in range(8):
                    ridx = e0 + 2 * j + sub
                    vals = plsc.load_gather(rows_v, [ridx, col])
                    wv = plsc.load_gather(w_v, [ridx])
                    plsc.store_scatter(rows_v, [ridx, col], jnp.abs(vals) * wv)
                return carry2
            lax.fori_loop(0, CHUNK // 16, vbody, 0)

        def fire_scatters(dst_v, rows_v, ssem):
            return [pltpu.async_copy(rows_v.at[pl.ds(g * G, G)],
                                     acc.at[dst_v.at[g]], ssem, add=True)
                    for g in range(GROUPS)]

        def pair_body(i, carry):
            cA = 2 * i
            cB = cA + 1
            eA = fire_edges(cA, src_v0, dst_v0, w_v0, esem0)
            eB = fire_edges(cB, src_v1, dst_v1, w_v1, esem1)
            for d in eA:
                d.wait()
            gA = fire_gathers(src_v0, rows_v0, gsem0)
            for d in eB:
                d.wait()
            gB = fire_gathers(src_v1, rows_v1, gsem1)
            for d in gA:
                d.wait()
            compute(w_v0, rows_v0)
            sA = fire_scatters(dst_v0, rows_v0, ssem0)
            for d in gB:
                d.wait()
            compute(w_v1, rows_v1)
            sB = fire_scatters(dst_v1, rows_v1, ssem1)
            for d in sA:
                d.wait()
            for d in sB:
                d.wait()
            return carry

        lax.fori_loop(0, NPAIR, pair_body, 0)

        plsc.subcore_barrier()
        pltpu.sync_copy(acc.at[pl.ds(sid * ROWS_PER_TILE, ROWS_PER_TILE)],
                        out_hbm.at[cid, pl.ds(sid * ROWS_PER_TILE, ROWS_PER_TILE)])

    return sc_kernel


_EPI_BLK = 2048


def _epi_body(ex_ref, in_ref, acc_ref, thr_ref, o_ref):
    a = acc_ref[...]
    gj = (a[0] + a[1]).T
    s = jnp.clip(ex_ref[...] - in_ref[...] + gj, -10.0, 10.0)
    o_ref[...] = jnp.maximum(s - jax.nn.softplus(thr_ref[...]), 0.0)


def _epilogue(ex_raw, in_raw, acc, thr):
    nspec = pl.BlockSpec((B, _EPI_BLK), lambda i: (0, i))
    return pl.pallas_call(
        _epi_body,
        out_shape=jax.ShapeDtypeStruct((B, N), jnp.float32),
        grid=(pl.cdiv(N, _EPI_BLK),),
        in_specs=[
            nspec,
            nspec,
            pl.BlockSpec((NC, _EPI_BLK, B), lambda i: (0, i, 0)),
            pl.BlockSpec((1, _EPI_BLK), lambda i: (0, i)),
        ],
        out_specs=nspec,
    )(ex_raw, in_raw, acc, thr)


@jax.jit
def kernel(ex_raw, in_raw, o_pre, gj_src, gj_dst, gj_w, threshold, decay):
    del decay
    o_t = o_pre.T
    src2 = gj_src.reshape(E // G, G)
    dst2 = gj_dst.reshape(E // G, G)
    zeros8 = jnp.zeros((N, B), jnp.float32)
    acc = _make_sc_kernel()(o_t, src2, dst2, gj_w, zeros8)

    return _epilogue(ex_raw, in_raw, acc, threshold.reshape(1, N))
